# baseline (speedup 1.0000x reference)
import functools

import jax
import jax.numpy as jnp
from jax import lax
from jax.experimental import pallas as pl
from jax.experimental.pallas import tpu as pltpu
from jax.experimental.pallas import tpu_sc as plsc

N = 10000
E = 320000
D = 128
R = 16
NC = 2
NS = 16
NW = NC * NS
EPW = E // NW
CHK = 80
NCHK = EPW // CHK
NP = 10240
ROWS_PT = NP // NS
BN = 1000
NXP = N + BN

_IOT = functools.partial(lax.broadcasted_iota, jnp.int32, (16,), 0)
_DNUMS = lax.GatherDimensionNumbers(
    offset_dims=(), collapsed_slice_dims=(0,), start_index_map=(0,))


def _mesh():
    return plsc.VectorSubcoreMesh(core_axis_name="c", subcore_axis_name="s")


def _wid():
    return lax.axis_index("s") * NC + lax.axis_index("c")


def _wexp_body(dst_hbm, rel_hbm, w_hbm, we_hbm, dst_v, rel_v, w_v, we_v):
    wid = _wid()
    iot = _IOT()

    def chunk(i, _):
        base = wid * EPW + i * CHK
        pltpu.sync_copy(dst_hbm.at[pl.ds(base, CHK)], dst_v)
        pltpu.sync_copy(rel_hbm.at[pl.ds(base, CHK)], rel_v)
        pltpu.sync_copy(w_hbm.at[dst_v], w_v)

        def pick(g, _):
            rel_vec = rel_v[pl.ds(g * 16, 16)]
            out = jnp.zeros((16,), jnp.float32)
            for j in range(16):
                e = g * 16 + j
                rj = rel_vec[j]
                widx = jnp.zeros((16,), jnp.int32) + rj
                w_vec = w_v[e, pl.ds(0, 16)]
                wj = lax.gather(w_vec, widx[:, None], _DNUMS, slice_sizes=(1,),
                                mode=lax.GatherScatterMode.PROMISE_IN_BOUNDS)
                out = jnp.where(iot == j, wj, out)
            we_v[pl.ds(g * 16, 16)] = out
            return 0
        lax.fori_loop(0, CHK // 16, pick, 0)
        pltpu.sync_copy(we_v, we_hbm.at[pl.ds(base, CHK)])
        return 0
    lax.fori_loop(0, NCHK, chunk, 0)


def _wexp(dst, rel, w_tab):
    f = functools.partial(
        pl.kernel,
        out_type=jax.ShapeDtypeStruct((8 * E,), jnp.float32),
        mesh=_mesh(),
        scratch_types=[
            pltpu.VMEM((CHK,), jnp.int32),
            pltpu.VMEM((CHK,), jnp.int32),
            pltpu.VMEM((CHK, D), jnp.float32),
            pltpu.VMEM((CHK,), jnp.float32),
        ],
    )(_wexp_body)
    return f(dst, rel, w_tab)


NPK = NP // 4
PK_PT = NPK // NS


def _gss_body(src_hbm, dst_hbm, rel_hbm, we_hbm, xp_hbm, acc_hbm,
              src_v, dst_v, rel_v, we_v, gidx_v, pidx_v, idx_v,
              rows_v, half_v, zbuf_v, acc_sh):
    cid = lax.axis_index("c")
    sid = lax.axis_index("s")
    wid = _wid()
    zeros16 = jnp.zeros((16,), jnp.float32)
    iot = _IOT()

    def zbfill(i, _):
        for j in range(8):
            zbuf_v[i, pl.ds(j * 16, 16)] = zeros16
        return 0
    lax.fori_loop(0, CHK, zbfill, 0)

    for p in range(4):
        off = 32 * p

        def zchunk(q, _):
            def mkz(g, _):
                idx_v[pl.ds(g * 16, 16)] = sid * PK_PT + q * CHK + g * 16 + iot
                return 0
            lax.fori_loop(0, CHK // 16, mkz, 0)
            pltpu.sync_copy(zbuf_v, acc_sh.at[idx_v])
            return 0
        lax.fori_loop(0, PK_PT // CHK, zchunk, 0)
        plsc.subcore_barrier()

        def chunk(i, _):
            base = wid * EPW + i * CHK
            pltpu.sync_copy(src_hbm.at[pl.ds(base, CHK)], src_v)
            pltpu.sync_copy(dst_hbm.at[pl.ds(base, CHK)], dst_v)
            pltpu.sync_copy(rel_hbm.at[pl.ds(base, CHK)], rel_v)
            pltpu.sync_copy(we_hbm.at[pl.ds(base, CHK)], we_v)

            def mkidx(g, _):
                s = src_v[pl.ds(g * 16, 16)]
                r = rel_v[pl.ds(g * 16, 16)]
                d = dst_v[pl.ds(g * 16, 16)]
                gidx_v[pl.ds(g * 16, 16)] = s * R + r
                pidx_v[pl.ds(g * 16, 16)] = d >> 2
                return 0
            lax.fori_loop(0, CHK // 16, mkidx, 0)

            pltpu.sync_copy(xp_hbm.at[gidx_v], rows_v)

            def scale(g, _):
                we_vec = we_v[pl.ds(g * 16, 16)]
                d_vec = dst_v[pl.ds(g * 16, 16)]
                for j in range(16):
                    e = g * 16 + j
                    wj = we_vec[j]
                    sub = d_vec[j] & 3
                    for c in range(2):
                        v = rows_v[e, pl.ds(off + c * 16, 16)] * wj
                        z = v * 0.0
                        for q in range(4):
                            half_v[e, pl.ds(q * 32 + c * 16, 16)] = (
                                jnp.where(sub == q, v, z))
                return 0
            lax.fori_loop(0, CHK // 16, scale, 0)

            pltpu.sync_copy(half_v, acc_sh.at[pidx_v], add=True)
            return 0
        lax.fori_loop(0, NCHK, chunk, 0)
        plsc.subcore_barrier()

        def rchunk(q, _):
            def mkz(g, _):
                idx_v[pl.ds(g * 16, 16)] = sid * PK_PT + q * CHK + g * 16 + iot
                return 0
            lax.fori_loop(0, CHK // 16, mkz, 0)
            pltpu.sync_copy(acc_sh.at[idx_v], half_v)
            pltpu.sync_copy(half_v,
                            acc_hbm.at[cid, p, pl.ds(sid * PK_PT + q * CHK, CHK)])
            return 0
        lax.fori_loop(0, PK_PT // CHK, rchunk, 0)
        plsc.subcore_barrier()


def _gss(src, dst, rel, we, xp):
    f = functools.partial(
        pl.kernel,
        out_type=jax.ShapeDtypeStruct((NC, 4, NPK, D), jnp.float32),
        mesh=_mesh(),
        scratch_types=[
            pltpu.VMEM((CHK,), jnp.int32),
            pltpu.VMEM((CHK,), jnp.int32),
            pltpu.VMEM((CHK,), jnp.int32),
            pltpu.VMEM((CHK,), jnp.float32),
            pltpu.VMEM((CHK,), jnp.int32),
            pltpu.VMEM((CHK,), jnp.int32),
            pltpu.VMEM((CHK,), jnp.int32),
            pltpu.VMEM((CHK, D), jnp.float32),
            pltpu.VMEM((CHK, D), jnp.float32),
            pltpu.VMEM((CHK, D), jnp.float32),
            pltpu.VMEM_SHARED((NPK, D), jnp.float32),
        ],
    )(_gss_body)
    return f(src, dst, rel, we, xp)


def _wcat_kernel(comp1_ref, basis1_ref, comp2_ref, basis2_ref,
                 wcat1_ref, wcat2_ref):
    for wcat_ref, comp_ref, basis_ref in ((wcat1_ref, comp1_ref, basis1_ref),
                                          (wcat2_ref, comp2_ref, basis2_ref)):
        for r in range(R):
            acc = comp_ref[r, 0] * basis_ref[0]
            for b in range(1, 10):
                acc = acc + comp_ref[r, b] * basis_ref[b]
            wcat_ref[:, r * D:(r + 1) * D] = acc


def _wcat(comp1, basis1, comp2, basis2):
    return pl.pallas_call(
        _wcat_kernel,
        grid=(1,),
        in_specs=[
            pl.BlockSpec(memory_space=pltpu.SMEM),
            pl.BlockSpec((10, D, D), lambda i: (0, 0, 0)),
            pl.BlockSpec(memory_space=pltpu.SMEM),
            pl.BlockSpec((10, D, D), lambda i: (0, 0, 0)),
        ],
        out_specs=[
            pl.BlockSpec((D, R * D), lambda i: (0, 0)),
            pl.BlockSpec((D, R * D), lambda i: (0, 0)),
        ],
        out_shape=[
            jax.ShapeDtypeStruct((D, R * D), jnp.float32),
            jax.ShapeDtypeStruct((D, R * D), jnp.float32),
        ],
    )(comp1, basis1, comp2, basis2)


def _prepw_kernel(cnt_ref, w_ref):
    c = cnt_ref[0, :, 0:16] + cnt_ref[1, :, 0:16]
    w = 1.0 / jnp.maximum(c, 1.0)
    w_ref[...] = jnp.concatenate([w, jnp.zeros((BN, D - 16), jnp.float32)], axis=1)


def _prepw(cnt):
    nb = N // BN
    return pl.pallas_call(
        _prepw_kernel,
        grid=(nb,),
        in_specs=[pl.BlockSpec((NC, BN, 32), lambda i: (0, i, 0))],
        out_specs=pl.BlockSpec((BN, D), lambda i: (i, 0)),
        out_shape=jax.ShapeDtypeStruct((N, D), jnp.float32),
    )(cnt)


def _xp_kernel(mode_ref, x_ref, wcat_ref, root_ref, xp_ref, xr_ref):
    x = x_ref[...]
    xpd = jnp.dot(x, wcat_ref[...], preferred_element_type=jnp.float32)
    col = lax.broadcasted_iota(jnp.int32, (BN, R * D), 1)
    pat = jnp.where(col % D == col // D, 1.0, 0.0).astype(jnp.float32)
    xp_ref[...] = jnp.where(mode_ref[0] == 0, pat, xpd)
    xr_ref[...] = jnp.dot(x, root_ref[...], preferred_element_type=jnp.float32)


def _xp(mode, x, wcat, root):
    nb = N // BN
    return pl.pallas_call(
        _xp_kernel,
        grid=(nb,),
        in_specs=[
            pl.BlockSpec(memory_space=pltpu.SMEM),
            pl.BlockSpec((BN, D), lambda i: (i, 0)),
            pl.BlockSpec((D, R * D), lambda i: (0, 0)),
            pl.BlockSpec((D, D), lambda i: (0, 0)),
        ],
        out_specs=[
            pl.BlockSpec((BN, R * D), lambda i: (i, 0)),
            pl.BlockSpec((BN, D), lambda i: (i, 0)),
        ],
        out_shape=[
            jax.ShapeDtypeStruct((N, R * D), jnp.float32),
            jax.ShapeDtypeStruct((N, D), jnp.float32),
        ],
    )(mode, x, wcat, root)


def _fin_kernel(acc_ref, xr_ref, bias_ref, out_ref):
    a = acc_ref[...]
    s = [a[0, p] + a[1, p] for p in range(4)]
    h = jnp.concatenate(s, axis=1) + xr_ref[...] + bias_ref[...]
    out_ref[...] = jnp.maximum(h, 0.0)


def _fin(acc, xr, bias):
    nb = N // BN
    return pl.pallas_call(
        _fin_kernel,
        grid=(nb,),
        in_specs=[
            pl.BlockSpec((NC, 4, BN, 32), lambda i: (0, 0, i, 0)),
            pl.BlockSpec((BN, D), lambda i: (i, 0)),
            pl.BlockSpec((1, D), lambda i: (0, 0)),
        ],
        out_specs=pl.BlockSpec((BN, D), lambda i: (i, 0)),
        out_shape=jax.ShapeDtypeStruct((N, D), jnp.float32),
    )(acc, xr, bias)


def kernel(x, edge_index, edge_type, basis1, comp1, root1, bias1,
           basis2, comp2, root2, bias2):
    src = edge_index[0]
    dst = edge_index[1]
    rel = edge_type

    wcat1, wcat2 = _wcat(comp1, basis1, comp2, basis2)
    wcats = jnp.stack([wcat1, wcat2])
    roots = jnp.stack([root1, root2])
    biases = jnp.stack([bias1.reshape(1, D), bias2.reshape(1, D)])

    v = wcat1[0, 0]
    niter = 3 + (v != v).astype(jnp.int32)

    def body(i, carry):
        h, xr, acc_prev, cnt_sv, we = carry
        layer = jnp.clip(i - 1, 0, 1)

        def upd_we(_):
            return _wexp(dst, rel, _prepw(cnt_sv))

        we = lax.cond(i == 1, upd_we, lambda _: we, None)

        bias = lax.dynamic_index_in_dim(biases, layer, keepdims=False)
        fin_out = _fin(acc_prev, xr, bias)
        h = jnp.where(i >= 2, fin_out, h)

        wcat = lax.dynamic_index_in_dim(wcats, layer, keepdims=False)
        root = lax.dynamic_index_in_dim(roots, layer, keepdims=False)
        xp, xr = _xp(i.reshape(1), h, wcat, root)

        acc = _gss(src, dst, rel, we, xp.reshape(N * R, D))
        acc = acc.reshape(NC, 4, NP, 32)
        cnt_sv = jnp.where(i == 0, acc[:, 0], cnt_sv)
        return (h, xr, acc, cnt_sv, we)

    init = (x,
            jnp.zeros((N, D), jnp.float32),
            jnp.zeros((NC, 4, NP, 32), jnp.float32),
            jnp.zeros((NC, NP, 32), jnp.float32),
            jnp.ones((8 * E,), jnp.float32))
    _, xr, acc, _, _ = lax.fori_loop(0, niter, body, init)
    return _fin(acc, xr, biases[1])


# speedup vs baseline: 1.1223x; 1.1223x over previous
import functools

import jax
import jax.numpy as jnp
from jax import lax
from jax.experimental import pallas as pl
from jax.experimental.pallas import tpu as pltpu
from jax.experimental.pallas import tpu_sc as plsc

N = 10000
E = 320000
D = 128
R = 16
NC = 2
NS = 16
NW = NC * NS
E2 = 327680
EPW = E2 // NW
CHK = 128
NCHK = EPW // CHK
ZCHK = 80
NP = 10240
ROWS_PT = NP // NS
BN = 1000
NXP = N + BN

_IOT = functools.partial(lax.broadcasted_iota, jnp.int32, (16,), 0)
_DNUMS = lax.GatherDimensionNumbers(
    offset_dims=(), collapsed_slice_dims=(0,), start_index_map=(0,))


def _mesh():
    return plsc.VectorSubcoreMesh(core_axis_name="c", subcore_axis_name="s")


def _wid():
    return lax.axis_index("s") * NC + lax.axis_index("c")


def _wexp_body(dst_hbm, rel_hbm, w_hbm, we_hbm, dst_v, rel_v, w_v, we_v):
    wid = _wid()
    iot = _IOT()

    def chunk(i, _):
        base = wid * EPW + i * CHK
        pltpu.sync_copy(dst_hbm.at[pl.ds(base, CHK)], dst_v)
        pltpu.sync_copy(rel_hbm.at[pl.ds(base, CHK)], rel_v)
        pltpu.sync_copy(w_hbm.at[dst_v], w_v)

        def pick(g, _):
            rel_vec = rel_v[pl.ds(g * 16, 16)]
            out = jnp.zeros((16,), jnp.float32)
            for j in range(16):
                e = g * 16 + j
                rj = rel_vec[j]
                widx = jnp.zeros((16,), jnp.int32) + rj
                w_vec = w_v[e, pl.ds(0, 16)]
                wj = lax.gather(w_vec, widx[:, None], _DNUMS, slice_sizes=(1,),
                                mode=lax.GatherScatterMode.PROMISE_IN_BOUNDS)
                out = jnp.where(iot == j, wj, out)
            we_v[pl.ds(g * 16, 16)] = out
            return 0
        lax.fori_loop(0, CHK // 16, pick, 0)
        pltpu.sync_copy(we_v, we_hbm.at[pl.ds(base, CHK)])
        return 0
    lax.fori_loop(0, NCHK, chunk, 0)


def _wexp(dst, rel, w_tab):
    f = functools.partial(
        pl.kernel,
        out_type=jax.ShapeDtypeStruct((E2,), jnp.float32),
        mesh=_mesh(),
        scratch_types=[
            pltpu.VMEM((CHK,), jnp.int32),
            pltpu.VMEM((CHK,), jnp.int32),
            pltpu.VMEM((CHK, D), jnp.float32),
            pltpu.VMEM((CHK,), jnp.float32),
        ],
    )(_wexp_body)
    return f(dst, rel, w_tab)


NPK = NP // 4
PK_PT = NPK // NS


def _gss_body(mode_hbm, edata_hbm, xp_hbm, acc_hbm,
              mode_v, ed_v, gidx_v, pidx_v, zidx_v,
              rows_v, half_v, zbuf_v, rbuf_v, acc_sh):
    cid = lax.axis_index("c")
    sid = lax.axis_index("s")
    wid = _wid()
    zeros16 = jnp.zeros((16,), jnp.float32)
    iot = _IOT()

    pltpu.sync_copy(mode_hbm, mode_v)
    mval = mode_v[pl.ds(0, 16)][0]

    def zbfill(i, _):
        for j in range(8):
            zbuf_v[i, pl.ds(j * 16, 16)] = zeros16
        return 0
    lax.fori_loop(0, ZCHK, zbfill, 0)

    for p in range(4):
        off = 32 * p
        act = (p == 0) | (mval > 0)

        def zchunk(q, _):
            def mkz(g, _):
                zidx_v[pl.ds(g * 16, 16)] = sid * PK_PT + q * ZCHK + g * 16 + iot
                return 0
            lax.fori_loop(0, ZCHK // 16, mkz, 0)
            pltpu.sync_copy(zbuf_v, acc_sh.at[zidx_v])
            return 0

        def zloop():
            lax.fori_loop(0, PK_PT // ZCHK, zchunk, 0)
        pl.when(act)(zloop)
        plsc.subcore_barrier()

        def chunk(i, _):
            base = wid * EPW + i * CHK
            pltpu.sync_copy(edata_hbm.at[:, pl.ds(base, CHK)], ed_v)

            def mkidx(g, _):
                s = ed_v[0, pl.ds(g * 16, 16)].astype(jnp.int32)
                d = ed_v[1, pl.ds(g * 16, 16)].astype(jnp.int32)
                r = ed_v[2, pl.ds(g * 16, 16)].astype(jnp.int32)
                gidx_v[pl.ds(g * 16, 16)] = s * R + r
                pidx_v[pl.ds(g * 16, 16)] = d >> 2
                return 0
            lax.fori_loop(0, CHK // 16, mkidx, 0)

            pltpu.sync_copy(xp_hbm.at[gidx_v], rows_v)

            def scale(g, _):
                we_vec = ed_v[3, pl.ds(g * 16, 16)]
                d_vec = ed_v[1, pl.ds(g * 16, 16)].astype(jnp.int32)
                for j in range(16):
                    e = g * 16 + j
                    wj = we_vec[j]
                    sub = d_vec[j] & 3
                    for c in range(2):
                        v = rows_v[e, pl.ds(off + c * 16, 16)] * wj
                        z = v * 0.0
                        for q in range(4):
                            half_v[e, pl.ds(q * 32 + c * 16, 16)] = (
                                jnp.where(sub == q, v, z))
                return 0
            lax.fori_loop(0, CHK // 16, scale, 0)

            pltpu.sync_copy(half_v, acc_sh.at[pidx_v], add=True)
            return 0

        def cloop():
            lax.fori_loop(0, NCHK, chunk, 0)
        pl.when(act)(cloop)
        plsc.subcore_barrier()

        def rchunk(q, _):
            def mkz(g, _):
                zidx_v[pl.ds(g * 16, 16)] = sid * PK_PT + q * ZCHK + g * 16 + iot
                return 0
            lax.fori_loop(0, ZCHK // 16, mkz, 0)
            pltpu.sync_copy(acc_sh.at[zidx_v], rbuf_v)
            pltpu.sync_copy(rbuf_v,
                            acc_hbm.at[cid, p, pl.ds(sid * PK_PT + q * ZCHK, ZCHK)])
            return 0

        def rloop():
            lax.fori_loop(0, PK_PT // ZCHK, rchunk, 0)
        pl.when(act)(rloop)
        plsc.subcore_barrier()


def _gss(mode, edata, xp):
    f = functools.partial(
        pl.kernel,
        out_type=jax.ShapeDtypeStruct((NC, 4, NPK, D), jnp.float32),
        mesh=_mesh(),
        scratch_types=[
            pltpu.VMEM((16,), jnp.int32),
            pltpu.VMEM((8, CHK), jnp.float32),
            pltpu.VMEM((CHK,), jnp.int32),
            pltpu.VMEM((CHK,), jnp.int32),
            pltpu.VMEM((ZCHK,), jnp.int32),
            pltpu.VMEM((CHK, D), jnp.float32),
            pltpu.VMEM((CHK, D), jnp.float32),
            pltpu.VMEM((ZCHK, D), jnp.float32),
            pltpu.VMEM((ZCHK, D), jnp.float32),
            pltpu.VMEM_SHARED((NPK, D), jnp.float32),
        ],
    )(_gss_body)
    return f(mode, edata, xp)


def _wcat_kernel(comp1_ref, basis1_ref, comp2_ref, basis2_ref,
                 wcat1_ref, wcat2_ref):
    for wcat_ref, comp_ref, basis_ref in ((wcat1_ref, comp1_ref, basis1_ref),
                                          (wcat2_ref, comp2_ref, basis2_ref)):
        for r in range(R):
            acc = comp_ref[r, 0] * basis_ref[0]
            for b in range(1, 10):
                acc = acc + comp_ref[r, b] * basis_ref[b]
            wcat_ref[:, r * D:(r + 1) * D] = acc


def _wcat(comp1, basis1, comp2, basis2):
    return pl.pallas_call(
        _wcat_kernel,
        grid=(1,),
        in_specs=[
            pl.BlockSpec(memory_space=pltpu.SMEM),
            pl.BlockSpec((10, D, D), lambda i: (0, 0, 0)),
            pl.BlockSpec(memory_space=pltpu.SMEM),
            pl.BlockSpec((10, D, D), lambda i: (0, 0, 0)),
        ],
        out_specs=[
            pl.BlockSpec((D, R * D), lambda i: (0, 0)),
            pl.BlockSpec((D, R * D), lambda i: (0, 0)),
        ],
        out_shape=[
            jax.ShapeDtypeStruct((D, R * D), jnp.float32),
            jax.ShapeDtypeStruct((D, R * D), jnp.float32),
        ],
    )(comp1, basis1, comp2, basis2)


def _prepw_kernel(cnt_ref, w_ref):
    c = cnt_ref[0, :, 0:16] + cnt_ref[1, :, 0:16]
    w = 1.0 / jnp.maximum(c, 1.0)
    w_ref[...] = jnp.concatenate([w, jnp.zeros((BNW, D - 16), jnp.float32)],
                                 axis=1)


BNW = 1024


def _prepw(cnt):
    nb = NP // BNW
    return pl.pallas_call(
        _prepw_kernel,
        grid=(nb,),
        in_specs=[pl.BlockSpec((NC, BNW, 32), lambda i: (0, i, 0))],
        out_specs=pl.BlockSpec((BNW, D), lambda i: (0, 0) if False else (i, 0)),
        out_shape=jax.ShapeDtypeStruct((NP, D), jnp.float32),
    )(cnt)


def _xp_kernel(mode_ref, x_ref, wcat_ref, root_ref, xp_ref, xr_ref):
    x = x_ref[...]
    xpd = jnp.dot(x, wcat_ref[...], preferred_element_type=jnp.float32)
    col = lax.broadcasted_iota(jnp.int32, (BN, R * D), 1)
    pat = jnp.where(col % D == col // D, 1.0, 0.0).astype(jnp.float32)
    xp_ref[...] = jnp.where(mode_ref[0] == 0, pat, xpd)
    xr_ref[...] = jnp.dot(x, root_ref[...], preferred_element_type=jnp.float32)


def _xp(mode, x, wcat, root):
    nb = N // BN
    return pl.pallas_call(
        _xp_kernel,
        grid=(nb,),
        in_specs=[
            pl.BlockSpec(memory_space=pltpu.SMEM),
            pl.BlockSpec((BN, D), lambda i: (i, 0)),
            pl.BlockSpec((D, R * D), lambda i: (0, 0)),
            pl.BlockSpec((D, D), lambda i: (0, 0)),
        ],
        out_specs=[
            pl.BlockSpec((BN, R * D), lambda i: (i, 0)),
            pl.BlockSpec((BN, D), lambda i: (i, 0)),
        ],
        out_shape=[
            jax.ShapeDtypeStruct((N, R * D), jnp.float32),
            jax.ShapeDtypeStruct((N, D), jnp.float32),
        ],
    )(mode, x, wcat, root)


def _fin_kernel(acc_ref, xr_ref, bias_ref, out_ref):
    a = acc_ref[...]
    s = [a[0, p] + a[1, p] for p in range(4)]
    h = jnp.concatenate(s, axis=1) + xr_ref[...] + bias_ref[...]
    out_ref[...] = jnp.maximum(h, 0.0)


def _fin(acc, xr, bias):
    nb = N // BN
    return pl.pallas_call(
        _fin_kernel,
        grid=(nb,),
        in_specs=[
            pl.BlockSpec((NC, 4, BN, 32), lambda i: (0, 0, i, 0)),
            pl.BlockSpec((BN, D), lambda i: (i, 0)),
            pl.BlockSpec((1, D), lambda i: (0, 0)),
        ],
        out_specs=pl.BlockSpec((BN, D), lambda i: (i, 0)),
        out_shape=jax.ShapeDtypeStruct((N, D), jnp.float32),
    )(acc, xr, bias)


def kernel(x, edge_index, edge_type, basis1, comp1, root1, bias1,
           basis2, comp2, root2, bias2):
    pad = E2 - E
    src = jnp.concatenate([edge_index[0], jnp.zeros((pad,), jnp.int32)])
    dst = jnp.concatenate([edge_index[1], jnp.full((pad,), N, jnp.int32)])
    rel = jnp.concatenate([edge_type, jnp.zeros((pad,), jnp.int32)])

    wcat1, wcat2 = _wcat(comp1, basis1, comp2, basis2)
    wcats = jnp.stack([wcat1, wcat2])
    roots = jnp.stack([root1, root2])
    biases = jnp.stack([bias1.reshape(1, D), bias2.reshape(1, D)])

    v = wcat1[0, 0]
    niter = 3 + (v != v).astype(jnp.int32)

    def body(i, carry):
        h, xr, acc_prev, cnt_sv, we = carry
        layer = jnp.clip(i - 1, 0, 1)

        def upd_we(_):
            return _wexp(dst, rel, _prepw(cnt_sv))

        we = lax.cond(i == 1, upd_we, lambda _: we, None)

        bias = lax.dynamic_index_in_dim(biases, layer, keepdims=False)
        fin_out = _fin(acc_prev, xr, bias)
        h = jnp.where(i >= 2, fin_out, h)

        wcat = lax.dynamic_index_in_dim(wcats, layer, keepdims=False)
        root = lax.dynamic_index_in_dim(roots, layer, keepdims=False)
        xp, xr = _xp(i.reshape(1), h, wcat, root)

        edata = jnp.stack([src.astype(jnp.float32), dst.astype(jnp.float32),
                           rel.astype(jnp.float32), we,
                           we, we, we, we])
        mode = jnp.full((16,), i, jnp.int32)
        acc = _gss(mode, edata, xp.reshape(N * R, D))
        acc = acc.reshape(NC, 4, NP, 32)
        cnt_sv = jnp.where(i == 0, acc[:, 0], cnt_sv)
        return (h, xr, acc, cnt_sv, we)

    init = (x,
            jnp.zeros((N, D), jnp.float32),
            jnp.zeros((NC, 4, NP, 32), jnp.float32),
            jnp.zeros((NC, NP, 32), jnp.float32),
            jnp.ones((E2,), jnp.float32))
    _, xr, acc, _, _ = lax.fori_loop(0, niter, body, init)
    return _fin(acc, xr, biases[1])
